# baseline (speedup 1.0000x reference)
import jax
import jax.numpy as jnp
from jax import lax
from jax.experimental import pallas as pl
from jax.experimental.pallas import tpu as pltpu

N = 32768
D = 512
B = 16
BN = 4096
NB = N // BN


def _reduce_kernel(x_ref, w_ref, b_ref, batch_ref, acc_ref, cnt_ref):
    s = pl.program_id(0)

    @pl.when(s == 0)
    def _init():
        acc_ref[...] = jnp.zeros_like(acc_ref)
        cnt_ref[...] = jnp.zeros_like(cnt_ref)

    bvec = batch_ref[0, 0, :]
    seg_iota = lax.broadcasted_iota(jnp.int32, (BN, B), 1)
    onehot = (bvec[:, None] == seg_iota).astype(jnp.float32)

    xb = x_ref[...]
    logit = jnp.sum(xb * w_ref[0, :][None, :], axis=1, keepdims=True) + b_ref[0]
    weighted = xb * jax.nn.sigmoid(logit)
    acc_ref[...] += jnp.dot(onehot.T, weighted, preferred_element_type=jnp.float32)
    cnt_ref[0, :] += jnp.sum(onehot, axis=0)


def _emit_kernel(x_ref, means_ref, batch_ref, out_ref):
    bvec = batch_ref[0, 0, :]
    seg_iota = lax.broadcasted_iota(jnp.int32, (BN, B), 1)
    onehot = (bvec[:, None] == seg_iota).astype(jnp.float32)
    out_ref[:, :D] = x_ref[...]
    out_ref[:, D:] = jnp.dot(onehot, means_ref[...],
                             preferred_element_type=jnp.float32)


def kernel(x, W, b, batch):
    batch32 = batch.astype(jnp.int32).reshape(NB, 1, BN)
    w_row = W.reshape(1, D)

    acc, cnt = pl.pallas_call(
        _reduce_kernel,
        grid=(NB,),
        in_specs=[
            pl.BlockSpec((BN, D), lambda s: (s, 0)),
            pl.BlockSpec((1, D), lambda s: (0, 0)),
            pl.BlockSpec(memory_space=pltpu.SMEM),
            pl.BlockSpec((1, 1, BN), lambda s: (s, 0, 0)),
        ],
        out_specs=[
            pl.BlockSpec((B, D), lambda s: (0, 0)),
            pl.BlockSpec((1, B), lambda s: (0, 0)),
        ],
        out_shape=[
            jax.ShapeDtypeStruct((B, D), jnp.float32),
            jax.ShapeDtypeStruct((1, B), jnp.float32),
        ],
    )(x, w_row, b, batch32)

    means = acc / jnp.maximum(cnt.reshape(B, 1), 1.0)

    out = pl.pallas_call(
        _emit_kernel,
        grid=(NB,),
        in_specs=[
            pl.BlockSpec((BN, D), lambda s: (s, 0)),
            pl.BlockSpec((B, D), lambda s: (0, 0)),
            pl.BlockSpec((1, 1, BN), lambda s: (s, 0, 0)),
        ],
        out_specs=pl.BlockSpec((BN, 2 * D), lambda s: (s, 0)),
        out_shape=jax.ShapeDtypeStruct((N, 2 * D), jnp.float32),
    )(x, means, batch32)
    return out
